# baseline (speedup 1.0000x reference)
import functools

import jax
import jax.numpy as jnp
from jax import lax
from jax.experimental import pallas as pl
from jax.experimental.pallas import tpu as pltpu
from jax.experimental.pallas import tpu_sc as plsc


def _gather_sparsecore(idx_flat, flat_table):
    n = idx_flat.shape[0]
    info = plsc.get_sparse_core_info()
    nc, ns = info.num_cores, info.num_subcores
    nw = nc * ns
    n_per_w = n // nw

    mesh = plsc.VectorSubcoreMesh(core_axis_name="c", subcore_axis_name="s")

    @functools.partial(
        pl.kernel,
        mesh=mesh,
        out_type=jax.ShapeDtypeStruct((n,), jnp.float32),
        scratch_types=[
            pltpu.VMEM((n_per_w,), jnp.int32),
            pltpu.VMEM((n_per_w,), jnp.float32),
            pltpu.SemaphoreType.DMA,
        ],
    )
    def gather_kernel(idx_hbm, table_hbm, out_hbm, idx_v, w_v, sem):
        wid = lax.axis_index("s") * nc + lax.axis_index("c")
        base = wid * n_per_w
        pltpu.sync_copy(idx_hbm.at[pl.ds(base, n_per_w)], idx_v)
        pltpu.async_copy(table_hbm.at[idx_v], w_v, sem).wait()
        pltpu.sync_copy(w_v, out_hbm.at[pl.ds(base, n_per_w)])

    return gather_kernel(idx_flat, flat_table)


def _fm_tensorcore(emb4, m2, g, h):
    rows, width = emb4.shape
    ncol = m2.shape[1]
    gpr = g.shape[1]
    blk = 512

    def body(emb_ref, m2_ref, g_ref, h_ref, out_ref):
        x = emb_ref[...]
        y = jnp.dot(x, m2_ref[...],
                    preferred_element_type=jnp.float32)
        ss = jnp.dot(y * y, h_ref[...],
                     preferred_element_type=jnp.float32)
        q = jnp.dot(x * x, g_ref[...],
                    preferred_element_type=jnp.float32)
        out_ref[...] = 0.5 * (ss - q)

    return pl.pallas_call(
        body,
        grid=(rows // blk,),
        in_specs=[
            pl.BlockSpec((blk, width), lambda i: (i, 0)),
            pl.BlockSpec((width, ncol), lambda i: (0, 0)),
            pl.BlockSpec((width, gpr), lambda i: (0, 0)),
            pl.BlockSpec((ncol, gpr), lambda i: (0, 0)),
        ],
        out_specs=pl.BlockSpec((blk, gpr), lambda i: (i, 0)),
        out_shape=jax.ShapeDtypeStruct((rows, gpr), jnp.float32),
    )(emb4, m2, g, h)


def _combine(dots, w4, s, bias11):
    rows, gpr = dots.shape
    wcols = w4.shape[1]

    def body(d_ref, w_ref, s_ref, b_ref, o_ref):
        lr = jnp.dot(w_ref[...], s_ref[...],
                     preferred_element_type=jnp.float32)
        o_ref[...] = d_ref[...] + lr + b_ref[0, 0]

    return pl.pallas_call(
        body,
        grid=(1,),
        in_specs=[
            pl.BlockSpec((rows, gpr), lambda i: (0, 0)),
            pl.BlockSpec((rows, wcols), lambda i: (0, 0)),
            pl.BlockSpec((wcols, gpr), lambda i: (0, 0)),
            pl.BlockSpec((1, 1), lambda i: (0, 0)),
        ],
        out_specs=pl.BlockSpec((rows, gpr), lambda i: (0, 0)),
        out_shape=jax.ShapeDtypeStruct((rows, gpr), jnp.float32),
    )(dots, w4, s, bias11)


def kernel(X, feature_emb, lr_table, bias):
    batch, nfields = X.shape
    vocab = lr_table.shape[1]
    d = feature_emb.shape[2]
    fd = nfields * d

    pack = 4
    rows = batch // pack
    width = pack * fd

    idx_flat = (
        X + jnp.arange(nfields, dtype=X.dtype)[None, :] * vocab
    ).reshape(-1)

    w_flat = _gather_sparsecore(idx_flat, lr_table.reshape(-1))

    eye_p = jnp.eye(pack, dtype=jnp.float32)
    eye_d = jnp.eye(d, dtype=jnp.float32)
    m2 = jnp.kron(eye_p, jnp.tile(eye_d, (nfields, 1)))
    g = jnp.kron(eye_p, jnp.ones((fd, 1), jnp.float32))
    h = jnp.kron(eye_p, jnp.ones((d, 1), jnp.float32))
    s = jnp.kron(eye_p, jnp.ones((nfields, 1), jnp.float32))

    dots = _fm_tensorcore(feature_emb.reshape(rows, width), m2, g, h)
    out4 = _combine(dots, w_flat.reshape(rows, pack * nfields), s,
                    bias.reshape(1, 1))
    return out4.reshape(batch, 1)


# speedup vs baseline: 2.0059x; 2.0059x over previous
import functools

import jax
import jax.numpy as jnp
from jax import lax
from jax.experimental import pallas as pl
from jax.experimental.pallas import tpu as pltpu
from jax.experimental.pallas import tpu_sc as plsc


def _gather_sparsecore(idx_flat, flat_table):
    n = idx_flat.shape[0]
    info = plsc.get_sparse_core_info()
    nc, ns = info.num_cores, info.num_subcores
    nw = nc * ns
    n_per_w = n // nw

    mesh = plsc.VectorSubcoreMesh(core_axis_name="c", subcore_axis_name="s")

    @functools.partial(
        pl.kernel,
        mesh=mesh,
        out_type=jax.ShapeDtypeStruct((n,), jnp.float32),
        scratch_types=[
            pltpu.VMEM((n_per_w,), jnp.int32),
            pltpu.VMEM((n_per_w,), jnp.float32),
            pltpu.SemaphoreType.DMA,
        ],
    )
    def gather_kernel(idx_hbm, table_hbm, out_hbm, idx_v, w_v, sem):
        wid = lax.axis_index("s") * nc + lax.axis_index("c")
        base = wid * n_per_w
        pltpu.sync_copy(idx_hbm.at[pl.ds(base, n_per_w)], idx_v)
        pltpu.async_copy(table_hbm.at[idx_v], w_v, sem).wait()
        pltpu.sync_copy(w_v, out_hbm.at[pl.ds(base, n_per_w)])

    return gather_kernel(idx_flat, flat_table)


def _fm_tensorcore(emb2d, sel, table_dummy):
    batch, fd = emb2d.shape
    d = sel.shape[1]
    blk = 1024

    def body(emb_ref, sel_ref, tab_ref, out_ref):
        x = emb_ref[...]
        sum_sq = jnp.sum(x * x, axis=1)
        s = jnp.dot(x, sel_ref[...],
                    preferred_element_type=jnp.float32)
        out_ref[...] = (0.5 * (jnp.sum(s * s, axis=1) - sum_sq))[:, None]

    return pl.pallas_call(
        body,
        grid=(batch // blk,),
        in_specs=[
            pl.BlockSpec((blk, fd), lambda i: (i, 0)),
            pl.BlockSpec((fd, d), lambda i: (0, 0)),
            pl.BlockSpec((1024,), lambda i: (0,)),
        ],
        out_specs=pl.BlockSpec((blk, 1), lambda i: (i, 0)),
        out_shape=jax.ShapeDtypeStruct((batch, 1), jnp.float32),
    )(emb2d, sel, table_dummy)


def _combine(dots, w2d, s, bias11):
    batch = dots.shape[0]
    nfields = w2d.shape[1]
    blk = 2048

    def body(d_ref, w_ref, s_ref, b_ref, o_ref):
        lr = jnp.dot(w_ref[...], s_ref[...],
                     preferred_element_type=jnp.float32)
        o_ref[...] = d_ref[...] + lr + b_ref[0, 0]

    return pl.pallas_call(
        body,
        grid=(batch // blk,),
        in_specs=[
            pl.BlockSpec((blk, 1), lambda i: (i, 0)),
            pl.BlockSpec((blk, nfields), lambda i: (i, 0)),
            pl.BlockSpec((nfields, 1), lambda i: (0, 0)),
            pl.BlockSpec((1, 1), lambda i: (0, 0)),
        ],
        out_specs=pl.BlockSpec((blk, 1), lambda i: (i, 0)),
        out_shape=jax.ShapeDtypeStruct((batch, 1), jnp.float32),
    )(dots, w2d, s, bias11)


def kernel(X, feature_emb, lr_table, bias):
    batch, nfields = X.shape
    vocab = lr_table.shape[1]
    d = feature_emb.shape[2]
    fd = nfields * d

    idx_flat = (
        X + jnp.arange(nfields, dtype=X.dtype)[None, :] * vocab
    ).reshape(-1)
    flat_table = lr_table.reshape(-1)

    w_flat = _gather_sparsecore(idx_flat, flat_table)

    sel = jnp.tile(jnp.eye(d, dtype=jnp.float32), (nfields, 1))
    dots = _fm_tensorcore(feature_emb.reshape(batch, fd), sel, flat_table)
    s = jnp.ones((nfields, 1), jnp.float32)
    out = _combine(dots, w_flat.reshape(batch, nfields), s,
                   bias.reshape(1, 1))
    return out


# speedup vs baseline: 2.2345x; 1.1139x over previous
import functools

import jax
import jax.numpy as jnp
from jax import lax
from jax.experimental import pallas as pl
from jax.experimental.pallas import tpu as pltpu
from jax.experimental.pallas import tpu_sc as plsc


def _gather_sparsecore(idx_flat, flat_table):
    n = idx_flat.shape[0]
    info = plsc.get_sparse_core_info()
    nc, ns = info.num_cores, info.num_subcores
    nw = nc * ns
    n_per_w = n // nw

    mesh = plsc.VectorSubcoreMesh(core_axis_name="c", subcore_axis_name="s")

    @functools.partial(
        pl.kernel,
        mesh=mesh,
        out_type=jax.ShapeDtypeStruct((n,), jnp.float32),
        scratch_types=[
            pltpu.VMEM((n_per_w,), jnp.int32),
            pltpu.VMEM((n_per_w,), jnp.float32),
            pltpu.SemaphoreType.DMA,
        ],
    )
    def gather_kernel(idx_hbm, table_hbm, out_hbm, idx_v, w_v, sem):
        wid = lax.axis_index("s") * nc + lax.axis_index("c")
        base = wid * n_per_w
        pltpu.sync_copy(idx_hbm.at[pl.ds(base, n_per_w)], idx_v)
        pltpu.async_copy(table_hbm.at[idx_v], w_v, sem).wait()
        pltpu.sync_copy(w_v, out_hbm.at[pl.ds(base, n_per_w)])

    return gather_kernel(idx_flat, flat_table)


def _fm_tensorcore(emb2d, sel, w_wide, k_sel, bias11):
    batch, fd = emb2d.shape
    d = sel.shape[1]
    wide = w_wide.shape[1]
    blk = 1024
    rpb = blk // 128

    def body(emb_ref, sel_ref, w_ref, k_ref, bias_ref, out_ref):
        x = emb_ref[...]
        sum_sq = jnp.sum(x * x, axis=1)
        s = jnp.dot(x, sel_ref[...],
                    preferred_element_type=jnp.float32)
        dots = 0.5 * (jnp.sum(s * s, axis=1) - sum_sq)
        lr8 = jnp.dot(w_ref[...], k_ref[...],
                      preferred_element_type=jnp.float32)
        lr = lr8.reshape(blk)
        out_ref[...] = (dots + lr + bias_ref[0, 0])[:, None]

    return pl.pallas_call(
        body,
        grid=(batch // blk,),
        in_specs=[
            pl.BlockSpec((blk, fd), lambda i: (i, 0)),
            pl.BlockSpec((fd, d), lambda i: (0, 0)),
            pl.BlockSpec((rpb, wide), lambda i: (i, 0)),
            pl.BlockSpec((wide, 128), lambda i: (0, 0)),
            pl.BlockSpec((1, 1), lambda i: (0, 0)),
        ],
        out_specs=pl.BlockSpec((blk, 1), lambda i: (i, 0)),
        out_shape=jax.ShapeDtypeStruct((batch, 1), jnp.float32),
    )(emb2d, sel, w_wide, k_sel, bias11)


def kernel(X, feature_emb, lr_table, bias):
    batch, nfields = X.shape
    vocab = lr_table.shape[1]
    d = feature_emb.shape[2]
    fd = nfields * d

    idx_flat = (
        X + jnp.arange(nfields, dtype=X.dtype)[None, :] * vocab
    ).reshape(-1)
    flat_table = lr_table.reshape(-1)

    w_flat = _gather_sparsecore(idx_flat, flat_table)

    sel = jnp.tile(jnp.eye(d, dtype=jnp.float32), (nfields, 1))
    wide = 128 * nfields
    k_sel = (jnp.arange(wide)[:, None] // nfields
             == jnp.arange(128)[None, :]).astype(jnp.float32)
    out = _fm_tensorcore(feature_emb.reshape(batch, fd), sel,
                         w_flat.reshape(batch // 128, wide), k_sel,
                         bias.reshape(1, 1))
    return out


# speedup vs baseline: 2.4394x; 1.0917x over previous
import functools

import jax
import jax.numpy as jnp
import numpy as np
from jax import lax
from jax.experimental import pallas as pl
from jax.experimental.pallas import tpu as pltpu
from jax.experimental.pallas import tpu_sc as plsc


def _gather_sparsecore(idx_flat, flat_table):
    n = idx_flat.shape[0]
    info = plsc.get_sparse_core_info()
    nc, ns = info.num_cores, info.num_subcores
    nw = nc * ns
    n_per_w = n // nw

    mesh = plsc.VectorSubcoreMesh(core_axis_name="c", subcore_axis_name="s")

    @functools.partial(
        pl.kernel,
        mesh=mesh,
        out_type=jax.ShapeDtypeStruct((n,), jnp.float32),
        scratch_types=[
            pltpu.VMEM((n_per_w,), jnp.int32),
            pltpu.VMEM((n_per_w,), jnp.float32),
            pltpu.SemaphoreType.DMA,
        ],
    )
    def gather_kernel(idx_hbm, table_hbm, out_hbm, idx_v, w_v, sem):
        wid = lax.axis_index("s") * nc + lax.axis_index("c")
        base = wid * n_per_w
        pltpu.sync_copy(idx_hbm.at[pl.ds(base, n_per_w)], idx_v)
        pltpu.async_copy(table_hbm.at[idx_v], w_v, sem).wait()
        pltpu.sync_copy(w_v, out_hbm.at[pl.ds(base, n_per_w)])

    return gather_kernel(idx_flat, flat_table)


def _fm_tensorcore(emb2d, sel, w_wide, k_sel, bias11):
    batch, fd = emb2d.shape
    d = sel.shape[1]
    wide = w_wide.shape[1]
    blk = 1024
    rpb = blk // 128

    def body(emb_ref, sel_ref, w_ref, k_ref, bias_ref, out_ref):
        x = emb_ref[...]
        sum_sq = jnp.sum(x * x, axis=1)
        s = jnp.dot(x, sel_ref[...],
                    preferred_element_type=jnp.float32)
        dots = 0.5 * (jnp.sum(s * s, axis=1) - sum_sq)
        lr8 = jnp.dot(w_ref[...], k_ref[...],
                      preferred_element_type=jnp.float32)
        lr = lr8.reshape(blk)
        out_ref[...] = (dots + lr + bias_ref[0, 0]).reshape(rpb, 128)

    return pl.pallas_call(
        body,
        grid=(batch // blk,),
        in_specs=[
            pl.BlockSpec((blk, fd), lambda i: (i, 0)),
            pl.BlockSpec((fd, d), lambda i: (0, 0)),
            pl.BlockSpec((rpb, wide), lambda i: (i, 0)),
            pl.BlockSpec((wide, 128), lambda i: (0, 0)),
            pl.BlockSpec((1, 1), lambda i: (0, 0)),
        ],
        out_specs=pl.BlockSpec((rpb, 128), lambda i: (i, 0)),
        out_shape=jax.ShapeDtypeStruct((batch // 128, 128), jnp.float32),
    )(emb2d, sel, w_wide, k_sel, bias11)


def kernel(X, feature_emb, lr_table, bias):
    batch, nfields = X.shape
    vocab = lr_table.shape[1]
    d = feature_emb.shape[2]
    fd = nfields * d

    n = batch * nfields
    field_off = (np.arange(n, dtype=np.int32) % nfields) * vocab
    idx_flat = X.reshape(-1) + jnp.asarray(field_off)
    flat_table = lr_table.reshape(-1)

    w_flat = _gather_sparsecore(idx_flat, flat_table)

    sel = jnp.asarray(
        np.tile(np.eye(d, dtype=np.float32), (nfields, 1)))
    wide = 128 * nfields
    k_sel = jnp.asarray(
        (np.arange(wide)[:, None] // nfields
         == np.arange(128)[None, :]).astype(np.float32))
    out128 = _fm_tensorcore(feature_emb.reshape(batch, fd), sel,
                            w_flat.reshape(batch // 128, wide), k_sel,
                            bias.reshape(1, 1))
    return out128.reshape(batch, 1)
